# baseline (speedup 1.0000x reference)
import functools

import jax
import jax.numpy as jnp
from jax import lax
from jax.experimental import pallas as pl
from jax.experimental.pallas import tpu as pltpu
from jax.experimental.pallas import tpu_sc as plsc

BATCH = 16384
VOCAB = 1000
EMB_DIM = 64
PADW = 128
EPS = 1e-5

_info = plsc.get_sparse_core_info()
_NC, _NS, _NL = _info.num_cores, _info.num_subcores, _info.num_lanes
_NW = _NC * _NS
_BPW = BATCH // _NW


def _prep_body(ages_ref, embt_ref, w_ref, b_ref, g_ref, bt_ref,
               p128_ref, s_ref):
    a = ages_ref[...]
    mean = jnp.mean(a)
    var = jnp.mean((a - mean) ** 2)
    scale = g_ref[0, 0] * lax.rsqrt(var + EPS)
    c0 = bt_ref[0, 0] - mean * scale

    ext = jnp.concatenate(
        [embt_ref[...], jnp.full((1, VOCAB), c0, dtype=jnp.float32)], axis=0)
    p0 = lax.dot_general(ext, w_ref[...], (((0,), (1,)), ((), ())),
                         preferred_element_type=jnp.float32)
    brow = jnp.pad(b_ref[...].reshape(1, EMB_DIM), ((0, 0), (0, PADW - EMB_DIM)))
    p128_ref[...] = jnp.pad(p0, ((0, 0), (0, PADW - EMB_DIM))) + brow

    row = lax.broadcasted_iota(jnp.int32, (EMB_DIM, PADW), 0)
    lane = lax.broadcasted_iota(jnp.int32, (EMB_DIM, PADW), 1)
    eye = jnp.where(row == lane, 1.0, 0.0)
    s_ref[...] = jnp.where(lane == EMB_DIM,
                           scale * w_ref[:, EMB_DIM:EMB_DIM + 1], eye)


_prep = pl.pallas_call(
    _prep_body,
    out_shape=[
        jax.ShapeDtypeStruct((VOCAB, PADW), jnp.float32),
        jax.ShapeDtypeStruct((EMB_DIM, PADW), jnp.float32),
    ],
)


_NCH = 2
_CH = _BPW // _NCH


def _sc_body(ids_hbm, ages_hbm, p128_hbm, g_hbm, idx_v, ages_v, rows_v,
             gsem, wsem, csem):
    wid = lax.axis_index("s") * _NC + lax.axis_index("c")
    base = wid * _BPW
    c_ids = pltpu.async_copy(ids_hbm.at[pl.ds(base, _BPW)], idx_v, csem)
    c_ages = pltpu.async_copy(ages_hbm.at[pl.ds(base, _BPW)], ages_v, csem)
    c_ids.wait()
    gathers = [
        pltpu.async_copy(p128_hbm.at[idx_v.at[pl.ds(k * _CH, _CH)]],
                         rows_v.at[pl.ds(k * _CH, _CH)], gsem.at[k])
        for k in range(_NCH)
    ]
    c_ages.wait()
    col_a = jnp.full((_NL,), EMB_DIM, dtype=jnp.int32)
    writes = []
    for k in range(_NCH):
        gathers[k].wait()

        def scat(t, carry):
            jbase = k * _CH + t * _NL
            a16 = ages_v[pl.ds(jbase, _NL)]
            row_idx = jbase + lax.iota(jnp.int32, _NL)
            plsc.store_scatter(rows_v, [row_idx, col_a], a16)
            return carry

        lax.fori_loop(0, _CH // _NL, scat, 0)
        writes.append(
            pltpu.async_copy(rows_v.at[pl.ds(k * _CH, _CH)],
                             g_hbm.at[pl.ds(base + k * _CH, _CH)], wsem))
    for w in writes:
        w.wait()


_sc_gather = functools.partial(
    pl.kernel,
    mesh=plsc.VectorSubcoreMesh(core_axis_name="c", subcore_axis_name="s"),
    out_type=jax.ShapeDtypeStruct((BATCH, PADW), jnp.float32),
    scratch_types=[
        pltpu.VMEM((_BPW,), jnp.int32),
        pltpu.VMEM((_BPW,), jnp.float32),
        pltpu.VMEM((_BPW, PADW), jnp.float32),
        pltpu.SemaphoreType.DMA((_NCH,)),
        pltpu.SemaphoreType.DMA,
        pltpu.SemaphoreType.DMA,
    ],
    compiler_params=pltpu.CompilerParams(use_tc_tiling_on_sc=True,
                                         needs_layout_passes=False,
                                         skip_device_barrier=True),
)(_sc_body)

_BLK = 4096


def _proj_body(s_ref, g_ref, out_ref):
    out_ref[...] = lax.dot_general(
        s_ref[...], g_ref[...], (((1,), (1,)), ((), ())),
        preferred_element_type=jnp.float32)


_proj = pl.pallas_call(
    _proj_body,
    grid=(BATCH // _BLK,),
    in_specs=[
        pl.BlockSpec((EMB_DIM, PADW), lambda i: (0, 0)),
        pl.BlockSpec((_BLK, PADW), lambda i: (i, 0)),
    ],
    out_specs=pl.BlockSpec((EMB_DIM, _BLK), lambda i: (0, i)),
    out_shape=jax.ShapeDtypeStruct((EMB_DIM, BATCH), jnp.float32),
)


def kernel(customer_ids, ages, emb_table, bn_gamma, bn_beta, W, b):
    ids = customer_ids.astype(jnp.int32)
    p128, s = _prep(ages.reshape(128, 128), emb_table.T, W, b,
                    bn_gamma.reshape(1, 1), bn_beta.reshape(1, 1))
    g = _sc_gather(ids, ages, p128)
    return _proj(s, g).T


# speedup vs baseline: 1.0184x; 1.0184x over previous
import functools

import jax
import jax.numpy as jnp
from jax import lax
from jax.experimental import pallas as pl
from jax.experimental.pallas import tpu as pltpu
from jax.experimental.pallas import tpu_sc as plsc

BATCH = 16384
VOCAB = 1000
EMB_DIM = 64
PADW = 128
EPS = 1e-5

_info = plsc.get_sparse_core_info()
_NC, _NS, _NL = _info.num_cores, _info.num_subcores, _info.num_lanes
_NW = _NC * _NS
_BPW = BATCH // _NW


def _prep_body(ages_ref, w_ref, b_ref, g_ref, bt_ref, s_ref):
    a = ages_ref[...]
    mean = jnp.mean(a)
    var = jnp.mean((a - mean) ** 2)
    scale = g_ref[0, 0] * lax.rsqrt(var + EPS)
    c0 = bt_ref[0, 0] - mean * scale

    wcol = w_ref[:, EMB_DIM:EMB_DIM + 1]
    full_lane = lax.broadcasted_iota(jnp.int32, (EMB_DIM, PADW), 1)
    w128 = jnp.pad(w_ref[...], ((0, 0), (0, PADW - (EMB_DIM + 1))))
    s_ref[...] = (w128 * jnp.where(full_lane == EMB_DIM, scale, 1.0)
                  + jnp.where(full_lane == EMB_DIM + 1,
                              b_ref[...] + c0 * wcol, 0.0))


_prep = pl.pallas_call(
    _prep_body,
    out_shape=jax.ShapeDtypeStruct((EMB_DIM, PADW), jnp.float32),
)

_NCH = 2
_CH = _BPW // _NCH


def _sc_body(ids_hbm, ages_hbm, emb128_hbm, g_hbm, idx_v, ages_v, rows_v,
             gsem, wsem, csem):
    wid = lax.axis_index("s") * _NC + lax.axis_index("c")
    base = wid * _BPW
    c_ids = pltpu.async_copy(ids_hbm.at[pl.ds(base, _BPW)], idx_v, csem)
    c_ages = pltpu.async_copy(ages_hbm.at[pl.ds(base, _BPW)], ages_v, csem)
    c_ids.wait()
    gathers = [
        pltpu.async_copy(emb128_hbm.at[idx_v.at[pl.ds(k * _CH, _CH)]],
                         rows_v.at[pl.ds(k * _CH, _CH)], gsem.at[k])
        for k in range(_NCH)
    ]
    c_ages.wait()
    col_a = jnp.full((_NL,), EMB_DIM, dtype=jnp.int32)
    col_1 = jnp.full((_NL,), EMB_DIM + 1, dtype=jnp.int32)
    ones = jnp.ones((_NL,), dtype=jnp.float32)
    writes = []
    for k in range(_NCH):
        gathers[k].wait()

        def scat(t, carry):
            jbase = k * _CH + t * _NL
            a16 = ages_v[pl.ds(jbase, _NL)]
            row_idx = jbase + lax.iota(jnp.int32, _NL)
            plsc.store_scatter(rows_v, [row_idx, col_a], a16)
            plsc.store_scatter(rows_v, [row_idx, col_1], ones)
            return carry

        lax.fori_loop(0, _CH // _NL, scat, 0)
        writes.append(
            pltpu.async_copy(rows_v.at[pl.ds(k * _CH, _CH)],
                             g_hbm.at[pl.ds(base + k * _CH, _CH)], wsem))
    for w in writes:
        w.wait()


_sc_gather = functools.partial(
    pl.kernel,
    mesh=plsc.VectorSubcoreMesh(core_axis_name="c", subcore_axis_name="s"),
    out_type=jax.ShapeDtypeStruct((BATCH, PADW), jnp.float32),
    scratch_types=[
        pltpu.VMEM((_BPW,), jnp.int32),
        pltpu.VMEM((_BPW,), jnp.float32),
        pltpu.VMEM((_BPW, PADW), jnp.float32),
        pltpu.SemaphoreType.DMA((_NCH,)),
        pltpu.SemaphoreType.DMA,
        pltpu.SemaphoreType.DMA,
    ],
    compiler_params=pltpu.CompilerParams(use_tc_tiling_on_sc=True,
                                         needs_layout_passes=False),
)(_sc_body)

_BLK = 4096


def _proj_body(s_ref, g_ref, out_ref):
    out_ref[...] = lax.dot_general(
        s_ref[...], g_ref[...], (((1,), (1,)), ((), ())),
        preferred_element_type=jnp.float32)


_proj = pl.pallas_call(
    _proj_body,
    grid=(BATCH // _BLK,),
    in_specs=[
        pl.BlockSpec((EMB_DIM, PADW), lambda i: (0, 0)),
        pl.BlockSpec((_BLK, PADW), lambda i: (i, 0)),
    ],
    out_specs=pl.BlockSpec((EMB_DIM, _BLK), lambda i: (0, i)),
    out_shape=jax.ShapeDtypeStruct((EMB_DIM, BATCH), jnp.float32),
)


def kernel(customer_ids, ages, emb_table, bn_gamma, bn_beta, W, b):
    ids = customer_ids.astype(jnp.int32)
    emb128 = jnp.pad(emb_table, ((0, 0), (0, PADW - EMB_DIM)))
    g = _sc_gather(ids, ages, emb128)
    s = _prep(ages.reshape(128, 128), W, b.reshape(EMB_DIM, 1),
              bn_gamma.reshape(1, 1), bn_beta.reshape(1, 1))
    return _proj(s, g).T
